# Initial kernel scaffold; baseline (speedup 1.0000x reference)
import jax
import jax.numpy as jnp
from jax.experimental import pallas as pl


def kernel(x, edge_index, edge_weight, W1, b1, gamma, beta, W2, b2):
    raise NotImplementedError("write your pallas kernel here")


# speedup vs baseline: 12.2468x; 12.2468x over previous
import functools

import jax
import jax.numpy as jnp
from jax import lax
from jax.experimental import pallas as pl
from jax.experimental.pallas import tpu as pltpu
from jax.experimental.pallas import tpu_sc as plsc

N = 10000
NPAD = 10240
F = 128
E = 320000
NW = 32
WIN = 128
WPW = 80
E_PAD = NW * WPW * WIN
NWIN = E_PAD // WIN
ROWS_PER_TILE = NPAD // 16


@functools.lru_cache(maxsize=None)
def _sc_kernels():
    mesh = plsc.VectorSubcoreMesh(core_axis_name="c", subcore_axis_name="s")
    deg = pl.kernel(
        _deg_body,
        out_type=jax.ShapeDtypeStruct((2, NPAD), jnp.float32),
        mesh=mesh,
        scratch_types=[
            pltpu.VMEM((1, WIN), jnp.int32),
            pltpu.VMEM((1, WIN), jnp.float32),
            pltpu.VMEM_SHARED((NPAD,), jnp.float32),
        ],
    )
    conv = pl.kernel(
        _conv_body,
        out_type=jax.ShapeDtypeStruct((2, NPAD, F), jnp.float32),
        mesh=mesh,
        scratch_types=[
            pltpu.VMEM((1, WIN), jnp.int32),
            pltpu.VMEM((1, WIN), jnp.int32),
            pltpu.VMEM((1, WIN), jnp.float32),
            pltpu.VMEM((WIN, F), jnp.float32),
            pltpu.VMEM_SHARED((NPAD, F), jnp.float32),
        ],
    )
    return deg, conv


def _deg_sc(cols2d, ew2d):
    return _sc_kernels()[0](cols2d, ew2d)


def _conv_sc(y, rows2d, cols2d, ew2d):
    return _sc_kernels()[1](y, rows2d, cols2d, ew2d)


def _deg_body(col_hbm, ew_hbm, out_hbm, colv, ewv, acc):
    c = lax.axis_index("c")
    s = lax.axis_index("s")

    @pl.loop(0, WIN, step=16)
    def _(i):
        ewv[0, pl.ds(i, 16)] = jnp.zeros((16,), jnp.float32)

    @pl.loop(0, ROWS_PER_TILE, step=WIN)
    def _(k):
        pltpu.sync_copy(ewv.at[0], acc.at[pl.ds(s * ROWS_PER_TILE + k, WIN)])

    plsc.subcore_barrier()

    wid = c * 16 + s

    @pl.loop(0, WPW)
    def _(k):
        w = wid * WPW + k
        pltpu.sync_copy(col_hbm.at[w], colv.at[0])
        pltpu.sync_copy(ew_hbm.at[w], ewv.at[0])
        pltpu.sync_copy(ewv.at[0], acc.at[colv.at[0]], add=True)

    plsc.subcore_barrier()
    pltpu.sync_copy(acc.at[pl.ds(s * ROWS_PER_TILE, ROWS_PER_TILE)],
                    out_hbm.at[c, pl.ds(s * ROWS_PER_TILE, ROWS_PER_TILE)])


def _conv_body(y_hbm, row_hbm, col_hbm, ew_hbm, out_hbm,
               rowv, colv, ewv, rows, acc):
    c = lax.axis_index("c")
    s = lax.axis_index("s")

    @pl.loop(0, WIN)
    def _(j):
        for f in range(F // 16):
            rows[j, pl.ds(f * 16, 16)] = jnp.zeros((16,), jnp.float32)

    @pl.loop(0, ROWS_PER_TILE, step=WIN)
    def _(k):
        pltpu.sync_copy(rows, acc.at[pl.ds(s * ROWS_PER_TILE + k, WIN)])

    plsc.subcore_barrier()

    wid = c * 16 + s

    @pl.loop(0, WPW)
    def _(k):
        w = wid * WPW + k
        pltpu.sync_copy(row_hbm.at[w], rowv.at[0])
        pltpu.sync_copy(col_hbm.at[w], colv.at[0])
        pltpu.sync_copy(ew_hbm.at[w], ewv.at[0])
        pltpu.sync_copy(y_hbm.at[rowv.at[0]], rows)

        @pl.loop(0, WIN, step=16)
        def _(g):
            wv = ewv[0, pl.ds(g, 16)]
            for i in range(16):
                wvi = jnp.full((16,), wv[i], dtype=jnp.float32)
                for f in range(F // 16):
                    sl = pl.ds(f * 16, 16)
                    rows[g + i, sl] = rows[g + i, sl] * wvi

        pltpu.sync_copy(rows, acc.at[colv.at[0]], add=True)

    plsc.subcore_barrier()
    pltpu.sync_copy(acc.at[pl.ds(s * ROWS_PER_TILE, ROWS_PER_TILE)],
                    out_hbm.at[c, pl.ds(s * ROWS_PER_TILE, ROWS_PER_TILE)])


_BLK = 1280
_NBLK = NPAD // _BLK


def _mm_body(x_ref, w_ref, o_ref):
    o_ref[...] = jnp.dot(x_ref[...], w_ref[...],
                         preferred_element_type=jnp.float32)


def _matmul_tc(x, w):
    return pl.pallas_call(
        _mm_body,
        out_shape=jax.ShapeDtypeStruct((NPAD, F), jnp.float32),
        grid=(_NBLK,),
        in_specs=[pl.BlockSpec((_BLK, F), lambda i: (i, 0)),
                  pl.BlockSpec((F, F), lambda i: (0, 0))],
        out_specs=pl.BlockSpec((_BLK, F), lambda i: (i, 0)),
    )(x, w)


def _scale_body(degt_ref, xw_ref, dinv_ref, y_ref):
    deg = degt_ref[:, 0:1] + degt_ref[:, 1:2] + 1.0
    dinv = jnp.where(deg > 0, lax.rsqrt(jnp.maximum(deg, 1e-12)), 0.0)
    dinv_ref[...] = dinv
    y_ref[...] = xw_ref[...] * dinv


def _scale_tc(degt, xw):
    return pl.pallas_call(
        _scale_body,
        out_shape=(jax.ShapeDtypeStruct((NPAD, 1), jnp.float32),
                   jax.ShapeDtypeStruct((NPAD, F), jnp.float32)),
        grid=(_NBLK,),
        in_specs=[pl.BlockSpec((_BLK, 2), lambda i: (i, 0)),
                  pl.BlockSpec((_BLK, F), lambda i: (i, 0))],
        out_specs=(pl.BlockSpec((_BLK, 1), lambda i: (i, 0)),
                   pl.BlockSpec((_BLK, F), lambda i: (i, 0))),
    )(degt, xw)


def _stats_body(p_ref, y_ref, dinv_ref, b_ref, h_ref, s_ref):
    i = pl.program_id(0)
    h = dinv_ref[...] * (p_ref[0] + p_ref[1] + y_ref[...]) + b_ref[...]
    h_ref[...] = h
    row = i * _BLK + lax.broadcasted_iota(jnp.int32, (_BLK, F), 0)
    hm = jnp.where(row < N, h, 0.0)

    @pl.when(i == 0)
    def _():
        s_ref[...] = jnp.zeros_like(s_ref)

    s_ref[0:1, :] += jnp.sum(hm, axis=0, keepdims=True)
    s_ref[1:2, :] += jnp.sum(hm * hm, axis=0, keepdims=True)


def _stats_tc(p, y, dinv, b):
    return pl.pallas_call(
        _stats_body,
        out_shape=(jax.ShapeDtypeStruct((NPAD, F), jnp.float32),
                   jax.ShapeDtypeStruct((2, F), jnp.float32)),
        grid=(_NBLK,),
        in_specs=[pl.BlockSpec((2, _BLK, F), lambda i: (0, i, 0)),
                  pl.BlockSpec((_BLK, F), lambda i: (i, 0)),
                  pl.BlockSpec((_BLK, 1), lambda i: (i, 0)),
                  pl.BlockSpec((1, F), lambda i: (0, 0))],
        out_specs=(pl.BlockSpec((_BLK, F), lambda i: (i, 0)),
                   pl.BlockSpec((2, F), lambda i: (0, 0))),
    )(p, y, dinv, b)


def _bn_body(h_ref, s_ref, g_ref, be_ref, w_ref, dinv_ref, y_ref):
    mean = s_ref[0:1, :] * (1.0 / N)
    var = s_ref[1:2, :] * (1.0 / N) - mean * mean
    rstd = lax.rsqrt(var + 1e-5)
    h = (h_ref[...] - mean) * rstd * g_ref[...] + be_ref[...]
    h = jnp.maximum(h, 0.0)
    y_ref[...] = jnp.dot(h, w_ref[...],
                         preferred_element_type=jnp.float32) * dinv_ref[...]


def _bn_tc(h, stats, gamma, beta, w, dinv):
    return pl.pallas_call(
        _bn_body,
        out_shape=jax.ShapeDtypeStruct((NPAD, F), jnp.float32),
        grid=(_NBLK,),
        in_specs=[pl.BlockSpec((_BLK, F), lambda i: (i, 0)),
                  pl.BlockSpec((2, F), lambda i: (0, 0)),
                  pl.BlockSpec((1, F), lambda i: (0, 0)),
                  pl.BlockSpec((1, F), lambda i: (0, 0)),
                  pl.BlockSpec((F, F), lambda i: (0, 0)),
                  pl.BlockSpec((_BLK, 1), lambda i: (i, 0))],
        out_specs=pl.BlockSpec((_BLK, F), lambda i: (i, 0)),
    )(h, stats, gamma, beta, w, dinv)


def _final_body(p_ref, y_ref, dinv_ref, b_ref, o_ref):
    o = dinv_ref[...] * (p_ref[0] + p_ref[1] + y_ref[...]) + b_ref[...]
    m = jnp.max(o, axis=1, keepdims=True)
    lse = m + jnp.log(jnp.sum(jnp.exp(o - m), axis=1, keepdims=True))
    o_ref[...] = o - lse


def _final_tc(p, y, dinv, b):
    return pl.pallas_call(
        _final_body,
        out_shape=jax.ShapeDtypeStruct((NPAD, F), jnp.float32),
        grid=(_NBLK,),
        in_specs=[pl.BlockSpec((2, _BLK, F), lambda i: (0, i, 0)),
                  pl.BlockSpec((_BLK, F), lambda i: (i, 0)),
                  pl.BlockSpec((_BLK, 1), lambda i: (i, 0)),
                  pl.BlockSpec((1, F), lambda i: (0, 0))],
        out_specs=pl.BlockSpec((_BLK, F), lambda i: (i, 0)),
    )(p, y, dinv, b)


def kernel(x, edge_index, edge_weight, W1, b1, gamma, beta, W2, b2):
    row = edge_index[0]
    col = edge_index[1]
    pad = E_PAD - E
    padidx = N + (jnp.arange(pad, dtype=jnp.int32) % (NPAD - N))
    rows2d = jnp.concatenate([row, padidx]).reshape(NWIN, WIN)
    cols2d = jnp.concatenate([col, padidx]).reshape(NWIN, WIN)
    ew2d = jnp.concatenate(
        [edge_weight, jnp.zeros((pad,), jnp.float32)]).reshape(NWIN, WIN)
    x_pad = jnp.concatenate([x, jnp.zeros((NPAD - N, F), jnp.float32)])

    deg_p = _deg_sc(cols2d, ew2d)
    xw1 = _matmul_tc(x_pad, W1)
    dinv, y1 = _scale_tc(jnp.transpose(deg_p), xw1)

    p1 = _conv_sc(y1, rows2d, cols2d, ew2d)
    h_pre, stats = _stats_tc(p1, y1, dinv, b1.reshape(1, F))
    y2 = _bn_tc(h_pre, stats, gamma.reshape(1, F), beta.reshape(1, F), W2,
                dinv)

    p2 = _conv_sc(y2, rows2d, cols2d, ew2d)
    out = _final_tc(p2, y2, dinv, b2.reshape(1, F))
    return out[:N]


# speedup vs baseline: 20.3143x; 1.6587x over previous
import dataclasses
import functools

import jax
import jax.numpy as jnp
from jax import lax
from jax.experimental import pallas as pl
from jax.experimental.pallas import tpu as pltpu
from jax.experimental.pallas import tpu_sc as plsc

N = 10000
NPAD = 10240
F = 128
E = 320000
NW = 32
WIN = 128
WPW = 80
E_PAD = NW * WPW * WIN
NWIN = E_PAD // WIN
ROWS_PER_TILE = NPAD // 16


NBUF = 2


@functools.lru_cache(maxsize=None)
def _sc_kernels():
    mesh = plsc.VectorSubcoreMesh(core_axis_name="c", subcore_axis_name="s")
    deg = pl.kernel(
        _deg_body,
        out_type=jax.ShapeDtypeStruct((2, NPAD), jnp.float32),
        mesh=mesh,
        scratch_types=[
            pltpu.VMEM((2, WIN), jnp.int32),
            pltpu.VMEM((2, WIN), jnp.float32),
            pltpu.VMEM_SHARED((NPAD,), jnp.float32),
            pltpu.SemaphoreType.DMA((2,)),
        ],
    )
    cp = pltpu.CompilerParams()
    if "needs_layout_passes" in pltpu.CompilerParams.__dataclass_fields__:
        cp = dataclasses.replace(cp, needs_layout_passes=False)
    conv = pl.kernel(
        _conv_body,
        out_type=jax.ShapeDtypeStruct((2, NPAD, F), jnp.float32),
        mesh=mesh,
        compiler_params=cp,
        scratch_types=[
            pltpu.VMEM((NBUF, 2, WIN), jnp.int32),
            pltpu.VMEM((NBUF, WIN), jnp.float32),
            pltpu.VMEM((NBUF, WIN, F), jnp.float32),
            pltpu.VMEM_SHARED((NPAD, F), jnp.float32),
            pltpu.SemaphoreType.DMA((NBUF,)),
            pltpu.SemaphoreType.DMA((NBUF,)),
        ],
    )
    return deg, conv


def _deg_sc(cols2d, ew2d):
    return _sc_kernels()[0](cols2d, ew2d)


def _conv_sc(y, rc3d, ew2d):
    return _sc_kernels()[1](y, rc3d, ew2d)


def _deg_body(col_hbm, ew_hbm, out_hbm, colv, ewv, acc, isem):
    c = lax.axis_index("c")
    s = lax.axis_index("s")

    @pl.loop(0, WIN, step=16)
    def _(i):
        ewv[0, pl.ds(i, 16)] = jnp.zeros((16,), jnp.float32)

    @pl.loop(0, ROWS_PER_TILE, step=WIN)
    def _(k):
        pltpu.sync_copy(ewv.at[0], acc.at[pl.ds(s * ROWS_PER_TILE + k, WIN)])

    plsc.subcore_barrier()

    base = (c * 16 + s) * WPW
    pltpu.async_copy(col_hbm.at[base], colv.at[0], isem.at[0])
    pltpu.async_copy(ew_hbm.at[base], ewv.at[0], isem.at[0])

    @pl.loop(0, WPW, step=2)
    def _(g):
        for b in range(2):
            w = base + g + b
            @pl.when(g + b + 1 < WPW)
            def _():
                pltpu.async_copy(col_hbm.at[w + 1], colv.at[1 - b],
                                 isem.at[1 - b])
                pltpu.async_copy(ew_hbm.at[w + 1], ewv.at[1 - b],
                                 isem.at[1 - b])

            pltpu.make_async_copy(col_hbm.at[base], colv.at[b],
                                  isem.at[b]).wait()
            pltpu.make_async_copy(ew_hbm.at[base], ewv.at[b],
                                  isem.at[b]).wait()
            pltpu.sync_copy(ewv.at[b], acc.at[colv.at[b]], add=True)

    plsc.subcore_barrier()
    pltpu.sync_copy(acc.at[pl.ds(s * ROWS_PER_TILE, ROWS_PER_TILE)],
                    out_hbm.at[c, pl.ds(s * ROWS_PER_TILE, ROWS_PER_TILE)])


def _conv_gather_start(y_hbm, rc_hbm, ew_hbm, rcbuf, ewbuf, rows, gsem, b, w):
    pltpu.sync_copy(rc_hbm.at[w], rcbuf.at[b])
    pltpu.sync_copy(ew_hbm.at[w], ewbuf.at[b])
    pltpu.async_copy(y_hbm.at[rcbuf.at[b, 0]], rows.at[b], gsem.at[b])


def _conv_body(y_hbm, rc_hbm, ew_hbm, out_hbm, rcbuf, ewbuf, rows, acc,
               gsem, ssem):
    c = lax.axis_index("c")
    s = lax.axis_index("s")

    @pl.loop(0, WIN)
    def _(j):
        for f in range(F // 16):
            rows[0, j, pl.ds(f * 16, 16)] = jnp.zeros((16,), jnp.float32)

    @pl.loop(0, ROWS_PER_TILE, step=WIN)
    def _(k):
        pltpu.sync_copy(rows.at[0], acc.at[pl.ds(s * ROWS_PER_TILE + k, WIN)])

    plsc.subcore_barrier()

    base = (c * 16 + s) * WPW

    for b in range(2):
        _conv_gather_start(y_hbm, rc_hbm, ew_hbm, rcbuf, ewbuf, rows, gsem,
                           b, base + b)

    @pl.loop(0, WPW, step=NBUF)
    def _(g):
        for b in range(NBUF):
            w = base + g + b
            pltpu.make_async_copy(y_hbm.at[pl.ds(0, WIN)], rows.at[b],
                                  gsem.at[b]).wait()

            @pl.loop(0, WIN, step=16)
            def _(g16):
                wv = ewbuf[b, pl.ds(g16, 16)]
                for i in range(16):
                    wvi = jnp.full((16,), wv[i], dtype=jnp.float32)
                    for f in range(F // 16):
                        sl = pl.ds(f * 16, 16)
                        rows[b, g16 + i, sl] = rows[b, g16 + i, sl] * wvi

            pltpu.async_copy(rows.at[b], acc.at[rcbuf.at[b, 1]], ssem.at[b],
                             add=True)
            pltpu.make_async_copy(rows.at[b], acc.at[pl.ds(0, WIN)],
                                  ssem.at[b]).wait()

            @pl.when(g + b + NBUF < WPW)
            def _():
                _conv_gather_start(y_hbm, rc_hbm, ew_hbm, rcbuf, ewbuf,
                                   rows, gsem, b, w + NBUF)

    plsc.subcore_barrier()
    pltpu.sync_copy(acc.at[pl.ds(s * ROWS_PER_TILE, ROWS_PER_TILE)],
                    out_hbm.at[c, pl.ds(s * ROWS_PER_TILE, ROWS_PER_TILE)])


_BLK = 1280
_NBLK = NPAD // _BLK


def _mm_body(x_ref, w_ref, o_ref):
    o_ref[...] = jnp.dot(x_ref[...], w_ref[...],
                         preferred_element_type=jnp.float32)


def _matmul_tc(x, w):
    return pl.pallas_call(
        _mm_body,
        out_shape=jax.ShapeDtypeStruct((NPAD, F), jnp.float32),
        grid=(_NBLK,),
        in_specs=[pl.BlockSpec((_BLK, F), lambda i: (i, 0)),
                  pl.BlockSpec((F, F), lambda i: (0, 0))],
        out_specs=pl.BlockSpec((_BLK, F), lambda i: (i, 0)),
    )(x, w)


def _scale_body(degt_ref, xw_ref, dinv_ref, y_ref):
    deg = degt_ref[:, 0:1] + degt_ref[:, 1:2] + 1.0
    dinv = jnp.where(deg > 0, lax.rsqrt(jnp.maximum(deg, 1e-12)), 0.0)
    dinv_ref[...] = dinv
    y_ref[...] = xw_ref[...] * dinv


def _scale_tc(degt, xw):
    return pl.pallas_call(
        _scale_body,
        out_shape=(jax.ShapeDtypeStruct((NPAD, 1), jnp.float32),
                   jax.ShapeDtypeStruct((NPAD, F), jnp.float32)),
        grid=(_NBLK,),
        in_specs=[pl.BlockSpec((_BLK, 2), lambda i: (i, 0)),
                  pl.BlockSpec((_BLK, F), lambda i: (i, 0))],
        out_specs=(pl.BlockSpec((_BLK, 1), lambda i: (i, 0)),
                   pl.BlockSpec((_BLK, F), lambda i: (i, 0))),
    )(degt, xw)


def _stats_body(p_ref, y_ref, dinv_ref, b_ref, h_ref, s_ref):
    i = pl.program_id(0)
    h = dinv_ref[...] * (p_ref[0] + p_ref[1] + y_ref[...]) + b_ref[...]
    h_ref[...] = h
    row = i * _BLK + lax.broadcasted_iota(jnp.int32, (_BLK, F), 0)
    hm = jnp.where(row < N, h, 0.0)

    @pl.when(i == 0)
    def _():
        s_ref[...] = jnp.zeros_like(s_ref)

    s_ref[0:1, :] += jnp.sum(hm, axis=0, keepdims=True)
    s_ref[1:2, :] += jnp.sum(hm * hm, axis=0, keepdims=True)


def _stats_tc(p, y, dinv, b):
    return pl.pallas_call(
        _stats_body,
        out_shape=(jax.ShapeDtypeStruct((NPAD, F), jnp.float32),
                   jax.ShapeDtypeStruct((2, F), jnp.float32)),
        grid=(_NBLK,),
        in_specs=[pl.BlockSpec((2, _BLK, F), lambda i: (0, i, 0)),
                  pl.BlockSpec((_BLK, F), lambda i: (i, 0)),
                  pl.BlockSpec((_BLK, 1), lambda i: (i, 0)),
                  pl.BlockSpec((1, F), lambda i: (0, 0))],
        out_specs=(pl.BlockSpec((_BLK, F), lambda i: (i, 0)),
                   pl.BlockSpec((2, F), lambda i: (0, 0))),
    )(p, y, dinv, b)


def _bn_body(h_ref, s_ref, g_ref, be_ref, w_ref, dinv_ref, y_ref):
    mean = s_ref[0:1, :] * (1.0 / N)
    var = s_ref[1:2, :] * (1.0 / N) - mean * mean
    rstd = lax.rsqrt(var + 1e-5)
    h = (h_ref[...] - mean) * rstd * g_ref[...] + be_ref[...]
    h = jnp.maximum(h, 0.0)
    y_ref[...] = jnp.dot(h, w_ref[...],
                         preferred_element_type=jnp.float32) * dinv_ref[...]


def _bn_tc(h, stats, gamma, beta, w, dinv):
    return pl.pallas_call(
        _bn_body,
        out_shape=jax.ShapeDtypeStruct((NPAD, F), jnp.float32),
        grid=(_NBLK,),
        in_specs=[pl.BlockSpec((_BLK, F), lambda i: (i, 0)),
                  pl.BlockSpec((2, F), lambda i: (0, 0)),
                  pl.BlockSpec((1, F), lambda i: (0, 0)),
                  pl.BlockSpec((1, F), lambda i: (0, 0)),
                  pl.BlockSpec((F, F), lambda i: (0, 0)),
                  pl.BlockSpec((_BLK, 1), lambda i: (i, 0))],
        out_specs=pl.BlockSpec((_BLK, F), lambda i: (i, 0)),
    )(h, stats, gamma, beta, w, dinv)


def _final_body(p_ref, y_ref, dinv_ref, b_ref, o_ref):
    o = dinv_ref[...] * (p_ref[0] + p_ref[1] + y_ref[...]) + b_ref[...]
    m = jnp.max(o, axis=1, keepdims=True)
    lse = m + jnp.log(jnp.sum(jnp.exp(o - m), axis=1, keepdims=True))
    o_ref[...] = o - lse


def _final_tc(p, y, dinv, b):
    return pl.pallas_call(
        _final_body,
        out_shape=jax.ShapeDtypeStruct((NPAD, F), jnp.float32),
        grid=(_NBLK,),
        in_specs=[pl.BlockSpec((2, _BLK, F), lambda i: (0, i, 0)),
                  pl.BlockSpec((_BLK, F), lambda i: (i, 0)),
                  pl.BlockSpec((_BLK, 1), lambda i: (i, 0)),
                  pl.BlockSpec((1, F), lambda i: (0, 0))],
        out_specs=pl.BlockSpec((_BLK, F), lambda i: (i, 0)),
    )(p, y, dinv, b)


def kernel(x, edge_index, edge_weight, W1, b1, gamma, beta, W2, b2):
    row = edge_index[0]
    col = edge_index[1]
    pad = E_PAD - E
    padidx = N + (jnp.arange(pad, dtype=jnp.int32) % (NPAD - N))
    rows2d = jnp.concatenate([row, padidx]).reshape(NWIN, WIN)
    cols2d = jnp.concatenate([col, padidx]).reshape(NWIN, WIN)
    ew2d = jnp.concatenate(
        [edge_weight, jnp.zeros((pad,), jnp.float32)]).reshape(NWIN, WIN)
    rc3d = jnp.stack([rows2d, cols2d], axis=1)
    x_pad = jnp.concatenate([x, jnp.zeros((NPAD - N, F), jnp.float32)])

    deg_p = _deg_sc(cols2d, ew2d)
    xw1 = _matmul_tc(x_pad, W1)
    dinv, y1 = _scale_tc(jnp.transpose(deg_p), xw1)

    p1 = _conv_sc(y1, rc3d, ew2d)
    h_pre, stats = _stats_tc(p1, y1, dinv, b1.reshape(1, F))
    y2 = _bn_tc(h_pre, stats, gamma.reshape(1, F), beta.reshape(1, F), W2,
                dinv)

    p2 = _conv_sc(y2, rc3d, ew2d)
    out = _final_tc(p2, y2, dinv, b2.reshape(1, F))
    return out[:N]


# speedup vs baseline: 29.6930x; 1.4617x over previous
import dataclasses
import functools

import jax
import jax.numpy as jnp
from jax import lax
from jax.experimental import pallas as pl
from jax.experimental.pallas import tpu as pltpu
from jax.experimental.pallas import tpu_sc as plsc

N = 10000
NPAD = 10240
F = 128
E = 320000
NW = 32
WIN = 112
WPW = 90
NWIN = 32 * WPW
E_PAD = NWIN * WIN
WIN_D = 128
WPW_D = 80
NWIN_D = 32 * WPW_D
E_PAD_D = NWIN_D * WIN_D
NQ = 3
ROWS_PER_TILE = NPAD // 16


@functools.lru_cache(maxsize=None)
def _sc_kernels():
    mesh = plsc.VectorSubcoreMesh(core_axis_name="c", subcore_axis_name="s")
    deg = pl.kernel(
        _deg_body,
        out_type=jax.ShapeDtypeStruct((2, NPAD), jnp.float32),
        mesh=mesh,
        scratch_types=[
            pltpu.VMEM((2, WIN_D), jnp.int32),
            pltpu.VMEM((2, WIN_D), jnp.float32),
            pltpu.VMEM_SHARED((NPAD,), jnp.float32),
            pltpu.SemaphoreType.DMA((2,)),
            pltpu.SemaphoreType.DMA((2,)),
        ],
    )
    cp = pltpu.CompilerParams()
    if "needs_layout_passes" in pltpu.CompilerParams.__dataclass_fields__:
        cp = dataclasses.replace(cp, needs_layout_passes=False)
    conv = pl.kernel(
        _conv_body,
        out_type=jax.ShapeDtypeStruct((2, NPAD, F), jnp.float32),
        mesh=mesh,
        compiler_params=cp,
        scratch_types=[
            pltpu.VMEM((NQ, WIN), jnp.int32),
            pltpu.VMEM((NQ, WIN), jnp.int32),
            pltpu.VMEM((NQ, WIN), jnp.float32),
            pltpu.VMEM((NQ, WIN, F), jnp.float32),
            pltpu.VMEM_SHARED((NPAD, F), jnp.float32),
            pltpu.SemaphoreType.DMA((NQ,)),
            pltpu.SemaphoreType.DMA((NQ,)),
            pltpu.SemaphoreType.DMA((NQ,)),
            pltpu.SemaphoreType.DMA((NQ,)),
        ],
    )
    return deg, conv


def _deg_sc(cols2d, ew2d):
    return _sc_kernels()[0](cols2d, ew2d)


def _conv_sc(y, rows2d, cols2d, ew2d):
    return _sc_kernels()[1](y, rows2d, cols2d, ew2d)


def _deg_body(col_hbm, ew_hbm, out_hbm, colv, ewv, acc, isem, dsem):
    c = lax.axis_index("c")
    s = lax.axis_index("s")

    @pl.loop(0, WIN_D, step=16)
    def _(i):
        ewv[0, pl.ds(i, 16)] = jnp.zeros((16,), jnp.float32)

    @pl.loop(0, ROWS_PER_TILE, step=WIN_D)
    def _(k):
        pltpu.sync_copy(ewv.at[0],
                        acc.at[pl.ds(s * ROWS_PER_TILE + k, WIN_D)])

    plsc.subcore_barrier()

    base = (c * 16 + s) * WPW_D
    pltpu.async_copy(col_hbm.at[base], colv.at[0], isem.at[0])
    pltpu.async_copy(ew_hbm.at[base], ewv.at[0], isem.at[0])

    @pl.loop(0, WPW_D, step=2)
    def _(g):
        for b in range(2):
            w = base + g + b
            @pl.when(g + b + 1 < WPW_D)
            def _():
                @pl.when(g + b > 0)
                def _():
                    pltpu.make_async_copy(ewv.at[1 - b],
                                          acc.at[pl.ds(0, WIN_D)],
                                          dsem.at[1 - b]).wait()
                pltpu.async_copy(col_hbm.at[w + 1], colv.at[1 - b],
                                 isem.at[1 - b])
                pltpu.async_copy(ew_hbm.at[w + 1], ewv.at[1 - b],
                                 isem.at[1 - b])

            pltpu.make_async_copy(col_hbm.at[base], colv.at[b],
                                  isem.at[b]).wait()
            pltpu.make_async_copy(ew_hbm.at[base], ewv.at[b],
                                  isem.at[b]).wait()
            pltpu.async_copy(ewv.at[b], acc.at[colv.at[b]], dsem.at[b],
                             add=True)

    for b in range(2):
        pltpu.make_async_copy(ewv.at[b], acc.at[pl.ds(0, WIN_D)],
                              dsem.at[b]).wait()

    plsc.subcore_barrier()
    pltpu.sync_copy(acc.at[pl.ds(s * ROWS_PER_TILE, ROWS_PER_TILE)],
                    out_hbm.at[c, pl.ds(s * ROWS_PER_TILE, ROWS_PER_TILE)])


def _re_issue(row_hbm, ew_hbm, rowb, ewb, isem, q, w):
    pltpu.async_copy(row_hbm.at[w], rowb.at[q], isem.at[q])
    pltpu.async_copy(ew_hbm.at[w], ewb.at[q], isem.at[q])


def _re_wait(row_hbm, ew_hbm, rowb, ewb, isem, q):
    pltpu.make_async_copy(row_hbm.at[0], rowb.at[q], isem.at[q]).wait()
    pltpu.make_async_copy(ew_hbm.at[0], ewb.at[q], isem.at[q]).wait()


def _conv_body(y_hbm, row_hbm, col_hbm, ew_hbm, out_hbm, rowb, colb, ewb,
               rows, acc, isem, csem, gsem, ssem):
    c = lax.axis_index("c")
    s = lax.axis_index("s")

    @pl.loop(0, WIN)
    def _(j):
        for f in range(F // 16):
            rows[0, j, pl.ds(f * 16, 16)] = jnp.zeros((16,), jnp.float32)

    @pl.loop(0, ROWS_PER_TILE, step=80)
    def _(k):
        pltpu.sync_copy(rows.at[0, pl.ds(0, 80)],
                        acc.at[pl.ds(s * ROWS_PER_TILE + k, 80)])

    plsc.subcore_barrier()

    base = (c * 16 + s) * WPW

    for q in range(NQ):
        _re_issue(row_hbm, ew_hbm, rowb, ewb, isem, q, base + q)
    for b in range(2):
        pltpu.sync_copy(col_hbm.at[base + b], colb.at[b])
        _re_wait(row_hbm, ew_hbm, rowb, ewb, isem, b)
        pltpu.async_copy(y_hbm.at[rowb.at[b]], rows.at[b], gsem.at[b])

    @pl.loop(0, WPW, step=NQ)
    def _(g):
        for b in range(NQ):
            w = base + g + b
            bn = (b + 2) % NQ
            pltpu.make_async_copy(y_hbm.at[pl.ds(0, WIN)], rows.at[b],
                                  gsem.at[b]).wait()

            @pl.loop(0, WIN, step=16)
            def _(g16):
                wv = ewb[b, pl.ds(g16, 16)]
                for i in range(16):
                    wvi = jnp.full((16,), wv[i], dtype=jnp.float32)
                    for f in range(F // 16):
                        sl = pl.ds(f * 16, 16)
                        rows[b, g16 + i, sl] = rows[b, g16 + i, sl] * wvi

            if b < 2:
                @pl.when(g > 0)
                def _():
                    pltpu.make_async_copy(col_hbm.at[0], colb.at[b],
                                          csem.at[b]).wait()
            else:
                pltpu.make_async_copy(col_hbm.at[0], colb.at[b],
                                      csem.at[b]).wait()

            pltpu.async_copy(rows.at[b], acc.at[colb.at[b]], ssem.at[b],
                             add=True)

            if b == 0:
                @pl.when(g > 0)
                def _():
                    pltpu.make_async_copy(rows.at[bn],
                                          acc.at[pl.ds(0, WIN)],
                                          ssem.at[bn]).wait()
            else:
                pltpu.make_async_copy(rows.at[bn], acc.at[pl.ds(0, WIN)],
                                      ssem.at[bn]).wait()

            @pl.when(g + b < WPW - NQ)
            def _():
                _re_issue(row_hbm, ew_hbm, rowb, ewb, isem, b, w + NQ)

            @pl.when(g + b < WPW - 2)
            def _():
                pltpu.async_copy(col_hbm.at[w + 2], colb.at[bn],
                                 csem.at[bn])
                _re_wait(row_hbm, ew_hbm, rowb, ewb, isem, bn)
                pltpu.async_copy(y_hbm.at[rowb.at[bn]], rows.at[bn],
                                 gsem.at[bn])

    pltpu.make_async_copy(rows.at[(WPW - 1) % NQ], acc.at[pl.ds(0, WIN)],
                          ssem.at[(WPW - 1) % NQ]).wait()

    plsc.subcore_barrier()
    pltpu.sync_copy(acc.at[pl.ds(s * ROWS_PER_TILE, ROWS_PER_TILE)],
                    out_hbm.at[c, pl.ds(s * ROWS_PER_TILE, ROWS_PER_TILE)])


_BLK = 1280
_NBLK = NPAD // _BLK


def _mm_body(x_ref, w_ref, o_ref):
    o_ref[...] = jnp.dot(x_ref[...], w_ref[...],
                         preferred_element_type=jnp.float32)


def _matmul_tc(x, w):
    return pl.pallas_call(
        _mm_body,
        out_shape=jax.ShapeDtypeStruct((NPAD, F), jnp.float32),
        grid=(_NBLK,),
        in_specs=[pl.BlockSpec((_BLK, F), lambda i: (i, 0)),
                  pl.BlockSpec((F, F), lambda i: (0, 0))],
        out_specs=pl.BlockSpec((_BLK, F), lambda i: (i, 0)),
    )(x, w)


def _scale_body(degt_ref, xw_ref, dinv_ref, y_ref):
    deg = degt_ref[:, 0:1] + degt_ref[:, 1:2] + 1.0
    dinv = jnp.where(deg > 0, lax.rsqrt(jnp.maximum(deg, 1e-12)), 0.0)
    dinv_ref[...] = dinv
    y_ref[...] = xw_ref[...] * dinv


def _scale_tc(degt, xw):
    return pl.pallas_call(
        _scale_body,
        out_shape=(jax.ShapeDtypeStruct((NPAD, 1), jnp.float32),
                   jax.ShapeDtypeStruct((NPAD, F), jnp.float32)),
        grid=(_NBLK,),
        in_specs=[pl.BlockSpec((_BLK, 2), lambda i: (i, 0)),
                  pl.BlockSpec((_BLK, F), lambda i: (i, 0))],
        out_specs=(pl.BlockSpec((_BLK, 1), lambda i: (i, 0)),
                   pl.BlockSpec((_BLK, F), lambda i: (i, 0))),
    )(degt, xw)


def _stats_body(p_ref, y_ref, dinv_ref, b_ref, h_ref, s_ref):
    i = pl.program_id(0)
    h = dinv_ref[...] * (p_ref[0] + p_ref[1] + y_ref[...]) + b_ref[...]
    h_ref[...] = h
    row = i * _BLK + lax.broadcasted_iota(jnp.int32, (_BLK, F), 0)
    hm = jnp.where(row < N, h, 0.0)

    @pl.when(i == 0)
    def _():
        s_ref[...] = jnp.zeros_like(s_ref)

    s_ref[0:1, :] += jnp.sum(hm, axis=0, keepdims=True)
    s_ref[1:2, :] += jnp.sum(hm * hm, axis=0, keepdims=True)


def _stats_tc(p, y, dinv, b):
    return pl.pallas_call(
        _stats_body,
        out_shape=(jax.ShapeDtypeStruct((NPAD, F), jnp.float32),
                   jax.ShapeDtypeStruct((2, F), jnp.float32)),
        grid=(_NBLK,),
        in_specs=[pl.BlockSpec((2, _BLK, F), lambda i: (0, i, 0)),
                  pl.BlockSpec((_BLK, F), lambda i: (i, 0)),
                  pl.BlockSpec((_BLK, 1), lambda i: (i, 0)),
                  pl.BlockSpec((1, F), lambda i: (0, 0))],
        out_specs=(pl.BlockSpec((_BLK, F), lambda i: (i, 0)),
                   pl.BlockSpec((2, F), lambda i: (0, 0))),
    )(p, y, dinv, b)


def _bn_body(h_ref, s_ref, g_ref, be_ref, w_ref, dinv_ref, y_ref):
    mean = s_ref[0:1, :] * (1.0 / N)
    var = s_ref[1:2, :] * (1.0 / N) - mean * mean
    rstd = lax.rsqrt(var + 1e-5)
    h = (h_ref[...] - mean) * rstd * g_ref[...] + be_ref[...]
    h = jnp.maximum(h, 0.0)
    y_ref[...] = jnp.dot(h, w_ref[...],
                         preferred_element_type=jnp.float32) * dinv_ref[...]


def _bn_tc(h, stats, gamma, beta, w, dinv):
    return pl.pallas_call(
        _bn_body,
        out_shape=jax.ShapeDtypeStruct((NPAD, F), jnp.float32),
        grid=(_NBLK,),
        in_specs=[pl.BlockSpec((_BLK, F), lambda i: (i, 0)),
                  pl.BlockSpec((2, F), lambda i: (0, 0)),
                  pl.BlockSpec((1, F), lambda i: (0, 0)),
                  pl.BlockSpec((1, F), lambda i: (0, 0)),
                  pl.BlockSpec((F, F), lambda i: (0, 0)),
                  pl.BlockSpec((_BLK, 1), lambda i: (i, 0))],
        out_specs=pl.BlockSpec((_BLK, F), lambda i: (i, 0)),
    )(h, stats, gamma, beta, w, dinv)


def _final_body(p_ref, y_ref, dinv_ref, b_ref, o_ref):
    o = dinv_ref[...] * (p_ref[0] + p_ref[1] + y_ref[...]) + b_ref[...]
    m = jnp.max(o, axis=1, keepdims=True)
    lse = m + jnp.log(jnp.sum(jnp.exp(o - m), axis=1, keepdims=True))
    o_ref[...] = o - lse


def _final_tc(p, y, dinv, b):
    return pl.pallas_call(
        _final_body,
        out_shape=jax.ShapeDtypeStruct((NPAD, F), jnp.float32),
        grid=(_NBLK,),
        in_specs=[pl.BlockSpec((2, _BLK, F), lambda i: (0, i, 0)),
                  pl.BlockSpec((_BLK, F), lambda i: (i, 0)),
                  pl.BlockSpec((_BLK, 1), lambda i: (i, 0)),
                  pl.BlockSpec((1, F), lambda i: (0, 0))],
        out_specs=pl.BlockSpec((_BLK, F), lambda i: (i, 0)),
    )(p, y, dinv, b)


def kernel(x, edge_index, edge_weight, W1, b1, gamma, beta, W2, b2):
    row = edge_index[0]
    col = edge_index[1]
    pad_d = E_PAD_D - E
    padidx_d = N + (jnp.arange(pad_d, dtype=jnp.int32) % (NPAD - N))
    pad_c = E_PAD - E
    padidx_c = padidx_d[:pad_c]
    x_pad = jnp.concatenate([x, jnp.zeros((NPAD - N, F), jnp.float32)])

    deg_p = _deg_sc(
        jnp.concatenate([col, padidx_d]).reshape(NWIN_D, WIN_D),
        jnp.concatenate(
            [edge_weight, jnp.zeros((pad_d,), jnp.float32)]
        ).reshape(NWIN_D, WIN_D))
    xw1 = _matmul_tc(x_pad, W1)
    dinv, y1 = _scale_tc(jnp.transpose(deg_p), xw1)

    rows2d = jnp.concatenate([row, padidx_c]).reshape(NWIN, WIN)
    cols2d = jnp.concatenate([col, padidx_c]).reshape(NWIN, WIN)
    ew2d = jnp.concatenate(
        [edge_weight, jnp.zeros((pad_c,), jnp.float32)]).reshape(NWIN, WIN)
    p1 = _conv_sc(y1, rows2d, cols2d, ew2d)
    h_pre, stats = _stats_tc(p1, y1, dinv, b1.reshape(1, F))
    y2 = _bn_tc(h_pre, stats, gamma.reshape(1, F), beta.reshape(1, F), W2,
                dinv)

    p2 = _conv_sc(y2, rows2d, cols2d, ew2d)
    out = _final_tc(p2, y2, dinv, b2.reshape(1, F))
    return out[:N]
